# baseline (speedup 1.0000x reference)
import functools

import jax
import jax.numpy as jnp
from jax import lax
from jax.experimental import pallas as pl
from jax.experimental.pallas import tpu as pltpu, tpu_sc as plsc

_VEC = 16
_CHUNK = 128


@functools.lru_cache(maxsize=None)
def _make_gather(V, D, B):
    info = plsc.get_sparse_core_info()
    nw = info.num_cores * info.num_subcores
    b_per_w = B // nw
    n_chunks = b_per_w // _CHUNK
    pair_w = 2 * D
    mesh = plsc.VectorSubcoreMesh(core_axis_name="c", subcore_axis_name="s")

    @functools.partial(
        pl.kernel,
        mesh=mesh,
        compiler_params=pltpu.CompilerParams(
            use_tc_tiling_on_sc=True, needs_layout_passes=False
        ),
        out_type=jax.ShapeDtypeStruct((D, B), jnp.float32),
        scratch_types=[
            pltpu.VMEM((b_per_w,), jnp.int32),
            pltpu.VMEM((b_per_w,), jnp.int32),
            pltpu.VMEM((2, _CHUNK, pair_w), jnp.float32),
            pltpu.VMEM((D, b_per_w), jnp.float32),
            pltpu.SemaphoreType.DMA,
        ],
    )
    def k(table_hbm, idx_hbm, out_hbm, idx_v, tid_v, pairs_v, cols_v, sem):
        wid = lax.axis_index("s") * info.num_cores + lax.axis_index("c")
        base = wid * b_per_w
        pltpu.sync_copy(idx_hbm.at[pl.ds(base, b_per_w)], idx_v)
        lanes = lax.iota(jnp.int32, _VEC)

        for v in range(b_per_w // _VEC):
            vec = idx_v[pl.ds(v * _VEC, _VEC)]
            tid_v[pl.ds(v * _VEC, _VEC)] = lax.shift_right_logical(vec, 1)

        def fire(c):
            return pltpu.async_copy(
                table_hbm.at[tid_v.at[pl.ds(c * _CHUNK, _CHUNK)]],
                pairs_v.at[c % 2],
                sem,
            )

        def select_chunk(c):
            buf = jnp.full((_VEC,), c % 2, jnp.int32)

            def sel(g, carry):
                j0 = c * _CHUNK + g * _VEC
                vec = idx_v[pl.ds(j0, _VEC)]
                half = lax.bitwise_and(vec, 1) * D
                j_rel = lanes + g * _VEC
                for d in range(D):
                    vals = plsc.load_gather(
                        pairs_v, [buf, j_rel, half + d]
                    )
                    cols_v[d, pl.ds(j0, _VEC)] = vals
                return carry

            lax.fori_loop(0, _CHUNK // _VEC, sel, 0)

        cp = fire(0)
        for c in range(n_chunks):
            cp_next = fire(c + 1) if c + 1 < n_chunks else None
            cp.wait()
            select_chunk(c)
            cp = cp_next
        pltpu.sync_copy(cols_v, out_hbm.at[:, pl.ds(base, b_per_w)])

    return k


@jax.jit
def kernel(source, hidden, cell, emb):
    V, D = emb.shape
    B = source.shape[0]
    table2 = emb.reshape(V // 2, 2 * D)
    out_t = _make_gather(V, D, B)(table2, source)
    return out_t.T


# speedup vs baseline: 2.6134x; 2.6134x over previous
import functools

import jax
import jax.numpy as jnp
from jax import lax
from jax.experimental import pallas as pl
from jax.experimental.pallas import tpu as pltpu, tpu_sc as plsc

_VEC = 16
_SUB = 8


@functools.lru_cache(maxsize=None)
def _make_gather(V, D, B):
    info = plsc.get_sparse_core_info()
    nw = info.num_cores * info.num_subcores
    b_per_w = B // nw
    n_vecs = b_per_w // _VEC
    mesh = plsc.VectorSubcoreMesh(core_axis_name="c", subcore_axis_name="s")

    @functools.partial(
        pl.kernel,
        mesh=mesh,
        compiler_params=pltpu.CompilerParams(
            use_tc_tiling_on_sc=True, needs_layout_passes=False
        ),
        out_type=jax.ShapeDtypeStruct((B, D), jnp.float32),
        scratch_types=[
            pltpu.VMEM((b_per_w,), jnp.int32),
            pltpu.VMEM((b_per_w, D), jnp.float32),
            pltpu.SemaphoreType.DMA,
        ],
    )
    def k(table_hbm, idx_hbm, out_hbm, idx_v, rows_v, sem):
        wid = lax.axis_index("s") * info.num_cores + lax.axis_index("c")
        base = wid * b_per_w
        pltpu.sync_copy(idx_hbm.at[pl.ds(base, b_per_w)], idx_v)
        lanes = lax.iota(jnp.int32, _VEC)

        def body(v, carry):
            vec = idx_v[pl.ds(v * _VEC, _VEC)]
            for i in range(_VEC):
                r = jnp.sum(jnp.where(lanes == i, vec, 0))
                t = lax.shift_right_logical(r, 3)
                s = lax.bitwise_and(r, _SUB - 1)
                pltpu.async_copy(
                    table_hbm.at[t, s], rows_v.at[v * _VEC + i], sem
                )
            return carry

        lax.fori_loop(0, n_vecs, body, 0)
        pltpu.make_async_copy(out_hbm.at[pl.ds(base, b_per_w)], rows_v, sem).wait()
        pltpu.sync_copy(rows_v, out_hbm.at[pl.ds(base, b_per_w)])

    return k


@jax.jit
def kernel(source, hidden, cell, emb):
    V, D = emb.shape
    B = source.shape[0]
    table3 = emb.reshape(V // _SUB, _SUB, D)
    return _make_gather(V, D, B)(table3, source)
